# baseline (speedup 1.0000x reference)
import functools

import jax
import jax.numpy as jnp
from jax import lax
from jax.experimental import pallas as pl
from jax.experimental.pallas import tpu as pltpu
from jax.experimental.pallas import tpu_sc as plsc

N = 32768
C = 512
NC = 2
NS = 16
NW = NC * NS
R = N // NW
L = 16
GCHUNK = 128
NG = R // GCHUNK


@functools.partial(
    pl.kernel,
    out_type=(
        jax.ShapeDtypeStruct((NW, L), jnp.float32),
        jax.ShapeDtypeStruct((NW, L), jnp.float32),
    ),
    mesh=plsc.VectorSubcoreMesh(core_axis_name="c", subcore_axis_name="s"),
    scratch_types=[
        pltpu.VMEM((R,), jnp.int32),
        pltpu.VMEM((R,), jnp.float32),
        pltpu.VMEM((R,), jnp.int32),
        pltpu.VMEM((R,), jnp.float32),
        pltpu.VMEM((L,), jnp.float32),
        pltpu.VMEM((L,), jnp.float32),
        pltpu.SemaphoreType.DMA,
    ],
)
def _masked_pick_sums(table_hbm, chord_hbm, mask_hbm, loss_out, cnt_out,
                      chord_v, mask_v, idx_v, picked_v, sum_v, cntv_v, sem):
    wid = lax.axis_index("s") * NC + lax.axis_index("c")
    base = wid * R

    pltpu.sync_copy(chord_hbm.at[pl.ds(base, R)], chord_v)
    pltpu.sync_copy(mask_hbm.at[pl.ds(base, R)], mask_v)

    for j in range(R // L):
        sl = pl.ds(j * L, L)
        rows = (base + j * L) + lax.iota(jnp.int32, 16)
        idx_v[sl] = rows * C + chord_v[sl]

    copies = [
        pltpu.async_copy(
            table_hbm.at[idx_v.at[pl.ds(g * GCHUNK, GCHUNK)]],
            picked_v.at[pl.ds(g * GCHUNK, GCHUNK)],
            sem,
        )
        for g in range(NG)
    ]
    for cp in copies:
        cp.wait()

    s = jnp.zeros((L,), jnp.float32)
    c = jnp.zeros((L,), jnp.float32)
    for j in range(R // L):
        sl = pl.ds(j * L, L)
        m = mask_v[sl]
        s = s + picked_v[sl] * m
        c = c + m
    sum_v[...] = s
    cntv_v[...] = c

    pltpu.sync_copy(sum_v, loss_out.at[wid])
    pltpu.sync_copy(cntv_v, cnt_out.at[wid])


def kernel(predict_chord, chord, mask):
    table = predict_chord.reshape(-1)
    m32 = mask.astype(jnp.float32)
    loss_parts, cnt_parts = _masked_pick_sums(table, chord, m32)
    return -jnp.sum(loss_parts) / jnp.sum(cnt_parts)


# speedup vs baseline: 2.7654x; 2.7654x over previous
import functools

import jax
import jax.numpy as jnp
from jax import lax
from jax.experimental import pallas as pl
from jax.experimental.pallas import tpu as pltpu
from jax.experimental.pallas import tpu_sc as plsc

N = 32768
C = 512
NC = 2
NS = 16
NW = NC * NS
R = N // NW
L = 16
GCHUNK = 128
NG = R // GCHUNK


@functools.partial(
    pl.kernel,
    out_type=(
        jax.ShapeDtypeStruct((NW, L), jnp.float32),
        jax.ShapeDtypeStruct((NW, L), jnp.float32),
    ),
    mesh=plsc.VectorSubcoreMesh(core_axis_name="c", subcore_axis_name="s"),
    scratch_types=[
        pltpu.VMEM((R,), jnp.int32),
        pltpu.VMEM((R,), jnp.float32),
        pltpu.VMEM((R,), jnp.int32),
        pltpu.VMEM((R,), jnp.float32),
        pltpu.VMEM((L,), jnp.float32),
        pltpu.VMEM((L,), jnp.float32),
        pltpu.SemaphoreType.DMA,
    ],
)
def _masked_pick_sums(table_hbm, chord_hbm, mask_hbm, loss_out, cnt_out,
                      chord_v, mask_v, idx_v, picked_v, sum_v, cntv_v, sem):
    wid = lax.axis_index("s") * NC + lax.axis_index("c")
    base = wid * R

    pltpu.sync_copy(chord_hbm.at[pl.ds(base, R)], chord_v)
    pltpu.sync_copy(mask_hbm.at[pl.ds(base, R)], mask_v)

    for j in range(R // L):
        sl = pl.ds(j * L, L)
        rows = (base + j * L) + lax.iota(jnp.int32, 16)
        c = chord_v[sl]
        idx_v[sl] = (
            ((rows >> 3) << 12)
            + ((c >> 7) << 10)
            + ((rows & 7) << 7)
            + (c & 127)
        )

    copies = [
        pltpu.async_copy(
            table_hbm.at[idx_v.at[pl.ds(g * GCHUNK, GCHUNK)]],
            picked_v.at[pl.ds(g * GCHUNK, GCHUNK)],
            sem,
        )
        for g in range(NG)
    ]
    for cp in copies:
        cp.wait()

    s = jnp.zeros((L,), jnp.float32)
    c = jnp.zeros((L,), jnp.float32)
    for j in range(R // L):
        sl = pl.ds(j * L, L)
        m = mask_v[sl]
        s = s + picked_v[sl] * m
        c = c + m
    sum_v[...] = s
    cntv_v[...] = c

    pltpu.sync_copy(sum_v, loss_out.at[wid])
    pltpu.sync_copy(cntv_v, cnt_out.at[wid])


def kernel(predict_chord, chord, mask):
    table = (
        predict_chord.reshape(N // 8, 8, C // 128, 128)
        .transpose(0, 2, 1, 3)
        .reshape(-1)
    )
    m32 = mask.astype(jnp.float32)
    loss_parts, cnt_parts = _masked_pick_sums(table, chord, m32)
    return -jnp.sum(loss_parts) / jnp.sum(cnt_parts)


# speedup vs baseline: 2.7962x; 1.0111x over previous
import functools

import jax
import jax.numpy as jnp
from jax import lax
from jax.experimental import pallas as pl
from jax.experimental.pallas import tpu as pltpu
from jax.experimental.pallas import tpu_sc as plsc

N = 32768
C = 512
NC = 2
NS = 16
NW = NC * NS
R = N // NW
L = 16
GCHUNK = 128
NG = R // GCHUNK


@functools.partial(
    pl.kernel,
    out_type=jax.ShapeDtypeStruct((NC, L), jnp.float32),
    mesh=plsc.VectorSubcoreMesh(core_axis_name="c", subcore_axis_name="s"),
    compiler_params=pltpu.CompilerParams(needs_layout_passes=False),
    scratch_types=[
        pltpu.VMEM((R,), jnp.int32),
        pltpu.VMEM((R,), jnp.float32),
        pltpu.VMEM((R,), jnp.int32),
        pltpu.VMEM((R,), jnp.float32),
        pltpu.VMEM((L,), jnp.float32),
        pltpu.VMEM((L,), jnp.float32),
        pltpu.VMEM((NS * L,), jnp.float32),
        pltpu.VMEM((NS * L,), jnp.float32),
        pltpu.VMEM((L,), jnp.float32),
        pltpu.VMEM_SHARED((NS * L,), jnp.float32),
        pltpu.VMEM_SHARED((NS * L,), jnp.float32),
        pltpu.SemaphoreType.DMA,
        pltpu.SemaphoreType.DMA,
    ],
)
def _sc_loss(table_hbm, chord_hbm, maskf_hbm, out_hbm,
             chord_v, maskf_v, idx_v, picked_v, sum_v, cnt_v,
             stile_v, ctile_v, out_v, shared_s, shared_c, sem, sem2):
    cid = lax.axis_index("c")
    sid = lax.axis_index("s")
    wid = sid * NC + cid
    base = wid * R

    cp1 = pltpu.async_copy(chord_hbm.at[pl.ds(base, R)], chord_v, sem2)
    cp2 = pltpu.async_copy(maskf_hbm.at[pl.ds(base, R)], maskf_v, sem2)
    cp1.wait()
    cp2.wait()

    iota = lax.iota(jnp.int32, L)
    gathers = []
    for g in range(NG):
        for t in range(GCHUNK // L):
            off = g * GCHUNK + t * L
            rows = (base + off) + iota
            c = chord_v[pl.ds(off, L)]
            idx_v[pl.ds(off, L)] = (
                ((rows >> 3) << 12)
                + ((c >> 7) << 10)
                + ((rows & 7) << 7)
                + (c & 127)
            )
        gathers.append(
            pltpu.async_copy(
                table_hbm.at[idx_v.at[pl.ds(g * GCHUNK, GCHUNK)]],
                picked_v.at[pl.ds(g * GCHUNK, GCHUNK)],
                sem,
            )
        )

    s = jnp.zeros((L,), jnp.float32)
    cnt = jnp.zeros((L,), jnp.float32)
    for g in range(NG):
        gathers[g].wait()
        for t in range(GCHUNK // L):
            off = g * GCHUNK + t * L
            m = maskf_v[pl.ds(off, L)]
            s = s + picked_v[pl.ds(off, L)] * m
            cnt = cnt + m
    sum_v[...] = s
    cnt_v[...] = cnt

    pltpu.sync_copy(sum_v, shared_s.at[pl.ds(sid * L, L)])
    pltpu.sync_copy(cnt_v, shared_c.at[pl.ds(sid * L, L)])
    plsc.subcore_barrier()

    @pl.when(sid == 0)
    def _():
        pltpu.sync_copy(shared_s, stile_v)
        pltpu.sync_copy(shared_c, ctile_v)
        ss = jnp.zeros((L,), jnp.float32)
        cc = jnp.zeros((L,), jnp.float32)
        for t in range(NS):
            ss = ss + stile_v[pl.ds(t * L, L)]
            cc = cc + ctile_v[pl.ds(t * L, L)]
        s_tot = jnp.sum(ss)
        c_tot = jnp.sum(cc)
        out_v[...] = jnp.where(
            iota == 0, s_tot, jnp.where(iota == 1, c_tot, 0.0))
        pltpu.sync_copy(out_v, out_hbm.at[cid])


def kernel(predict_chord, chord, mask):
    table = (
        predict_chord.reshape(N // 8, 8, C // 128, 128)
        .transpose(0, 2, 1, 3)
        .reshape(-1)
    )
    maskf = mask.astype(jnp.float32)
    parts = _sc_loss(table, chord, maskf)
    s = parts[0, 0] + parts[1, 0]
    cnt = parts[0, 1] + parts[1, 1]
    return -s / cnt


# speedup vs baseline: 3.2111x; 1.1484x over previous
import functools

import jax
import jax.numpy as jnp
from jax import lax
from jax.experimental import pallas as pl
from jax.experimental.pallas import tpu as pltpu
from jax.experimental.pallas import tpu_sc as plsc

N = 32768
C = 512
NC = 2
NS = 16
NW = NC * NS
R = N // NW
L = 16
GCHUNK = 128
NG = R // GCHUNK


@functools.partial(
    pl.kernel,
    out_type=jax.ShapeDtypeStruct((N,), jnp.float32),
    mesh=plsc.VectorSubcoreMesh(core_axis_name="c", subcore_axis_name="s"),
    compiler_params=pltpu.CompilerParams(needs_layout_passes=False),
    scratch_types=[
        pltpu.VMEM((R,), jnp.int32),
        pltpu.VMEM((R,), jnp.int32),
        pltpu.VMEM((R,), jnp.float32),
        pltpu.SemaphoreType.DMA,
        pltpu.SemaphoreType.DMA,
    ],
)
def _sc_gather(table_hbm, chord_hbm, out_hbm, chord_v, idx_v, picked_v,
               sem, sem2):
    cid = lax.axis_index("c")
    sid = lax.axis_index("s")
    wid = sid * NC + cid
    base = wid * R

    pltpu.sync_copy(chord_hbm.at[pl.ds(base, R)], chord_v)

    iota = lax.iota(jnp.int32, L)
    gathers = []
    for g in range(NG):
        for t in range(GCHUNK // L):
            off = g * GCHUNK + t * L
            rows = (base + off) + iota
            c = chord_v[pl.ds(off, L)]
            idx_v[pl.ds(off, L)] = (
                ((rows >> 3) << 12)
                + ((c >> 7) << 10)
                + ((rows & 7) << 7)
                + (c & 127)
            )
        gathers.append(
            pltpu.async_copy(
                table_hbm.at[idx_v.at[pl.ds(g * GCHUNK, GCHUNK)]],
                picked_v.at[pl.ds(g * GCHUNK, GCHUNK)],
                sem,
            )
        )

    writes = []
    for g in range(NG):
        gathers[g].wait()
        writes.append(
            pltpu.async_copy(
                picked_v.at[pl.ds(g * GCHUNK, GCHUNK)],
                out_hbm.at[pl.ds(base + g * GCHUNK, GCHUNK)],
                sem2,
            )
        )
    for w in writes:
        w.wait()


def _masked_mean_body(picked_ref, mask_ref, out_ref):
    p = picked_ref[...]
    m = mask_ref[...].astype(jnp.float32)
    s = jnp.sum(p * m)
    c = jnp.sum(m)
    out_ref[0, 0] = -s / c


_masked_mean = pl.pallas_call(
    _masked_mean_body,
    out_shape=jax.ShapeDtypeStruct((1, 1), jnp.float32),
    out_specs=pl.BlockSpec(memory_space=pltpu.SMEM),
)


def kernel(predict_chord, chord, mask):
    table = (
        predict_chord.reshape(N // 8, 8, C // 128, 128)
        .transpose(0, 2, 1, 3)
        .reshape(-1)
    )
    picked = _sc_gather(table, chord)
    loss = _masked_mean(picked.reshape(N // 128, 128),
                        mask.reshape(N // 128, 128))
    return loss[0, 0]


# speedup vs baseline: 3.2471x; 1.0112x over previous
import functools

import jax
import jax.numpy as jnp
from jax import lax
from jax.experimental import pallas as pl
from jax.experimental.pallas import tpu as pltpu
from jax.experimental.pallas import tpu_sc as plsc

N = 32768
C = 512
NC = 2
NS = 16
NW = NC * NS
R = N // NW
L = 16
GCHUNK = 128
NG = R // GCHUNK


@functools.partial(
    pl.kernel,
    out_type=jax.ShapeDtypeStruct((N,), jnp.float32),
    mesh=plsc.VectorSubcoreMesh(core_axis_name="c", subcore_axis_name="s"),
    compiler_params=pltpu.CompilerParams(needs_layout_passes=False),
    scratch_types=[
        pltpu.VMEM((R,), jnp.int32),
        pltpu.VMEM((R,), jnp.int32),
        pltpu.VMEM((R,), jnp.float32),
        pltpu.SemaphoreType.DMA,
        pltpu.SemaphoreType.DMA,
    ],
)
def _sc_gather(table_hbm, chord_hbm, out_hbm, chord_v, idx_v, picked_v,
               sem, sem2):
    cid = lax.axis_index("c")
    sid = lax.axis_index("s")
    wid = sid * NC + cid
    base = wid * R

    pltpu.sync_copy(chord_hbm.at[pl.ds(base, R)], chord_v)

    iota = lax.iota(jnp.int32, L)
    lanes = ((iota >> 3) << 12) + ((iota & 7) << 7)

    def idx_body(t, _):
        off = pl.multiple_of(t * L, L)
        c = chord_v[pl.ds(off, L)]
        rb = (base + t * L) >> 3
        idx_v[pl.ds(off, L)] = (
            ((rb << 12) + lanes) + (((c >> 7) << 10) + (c & 127))
        )
        return 0

    lax.fori_loop(0, R // L, idx_body, 0, unroll=2)

    gathers = []
    for g in range(NG):
        gathers.append(
            pltpu.async_copy(
                table_hbm.at[idx_v.at[pl.ds(g * GCHUNK, GCHUNK)]],
                picked_v.at[pl.ds(g * GCHUNK, GCHUNK)],
                sem,
            )
        )

    writes = []
    for g in range(NG):
        gathers[g].wait()
        writes.append(
            pltpu.async_copy(
                picked_v.at[pl.ds(g * GCHUNK, GCHUNK)],
                out_hbm.at[pl.ds(base + g * GCHUNK, GCHUNK)],
                sem2,
            )
        )
    for w in writes:
        w.wait()


def _masked_mean_body(picked_ref, mask_ref, out_ref):
    p = picked_ref[...]
    m = mask_ref[...].astype(jnp.float32)
    s = jnp.sum(p * m)
    c = jnp.sum(m)
    out_ref[0, 0] = -s / c


_masked_mean = pl.pallas_call(
    _masked_mean_body,
    out_shape=jax.ShapeDtypeStruct((1, 1), jnp.float32),
    out_specs=pl.BlockSpec(memory_space=pltpu.SMEM),
)


def kernel(predict_chord, chord, mask):
    table = (
        predict_chord.reshape(N // 8, 8, C // 128, 128)
        .transpose(0, 2, 1, 3)
        .reshape(-1)
    )
    picked = _sc_gather(table, chord)
    loss = _masked_mean(picked.reshape(N // 128, 128),
                        mask.reshape(N // 128, 128))
    return loss[0, 0]
